# baseline (device time: 294458 ns/iter reference)
import jax
import jax.numpy as jnp
from jax import lax
from jax.experimental import pallas as pl
from jax.experimental.pallas import tpu as pltpu

N_DEV = 32
N_STEPS = 2 * (N_DEV - 1)
K_CHUNKS = 4


def kernel(x, Wg, Wu, Wd):
    m, d = x.shape
    h_per = Wg.shape[1]
    chunk = m // N_DEV
    kc = h_per // K_CHUNKS

    def body(x_ref, wg_ref, wu_ref, wd_ref, out_ref,
             recv_buf, send_sems, recv_sems, credit_sem):
        my = lax.axis_index("i")
        left = lax.rem(my + N_DEV - 1, N_DEV)
        right = lax.rem(my + 1, N_DEV)

        barrier_sem = pltpu.get_barrier_semaphore()
        pl.semaphore_signal(barrier_sem, inc=1, device_id=(left,),
                            device_id_type=pl.DeviceIdType.MESH)
        pl.semaphore_signal(barrier_sem, inc=1, device_id=(right,),
                            device_id_type=pl.DeviceIdType.MESH)
        pl.semaphore_wait(barrier_sem, 2)

        for k in range(K_CHUNKS):
            sl = slice(k * kc, (k + 1) * kc)
            g = jnp.dot(x_ref[...], wg_ref[:, sl],
                        preferred_element_type=jnp.float32)
            u = jnp.dot(x_ref[...], wu_ref[:, sl],
                        preferred_element_type=jnp.float32)
            hblk = g * (u / (1.0 + jnp.exp(-u)))
            p = jnp.dot(hblk, wd_ref[sl, :], preferred_element_type=jnp.float32)
            if k == 0:
                out_ref[...] = p
            else:
                out_ref[...] = out_ref[...] + p

        def step(s, carry):
            slot = lax.rem(s, 2)
            send_c = lax.rem(my - s + 3 * N_DEV, N_DEV)
            recv_c = lax.rem(my - s - 1 + 3 * N_DEV, N_DEV)

            @pl.when(s >= 2)
            def _():
                pl.semaphore_wait(credit_sem, 1)

            rdma = pltpu.make_async_remote_copy(
                src_ref=out_ref.at[pl.ds(send_c * chunk, chunk), :],
                dst_ref=recv_buf.at[slot],
                send_sem=send_sems.at[slot],
                recv_sem=recv_sems.at[slot],
                device_id=(right,),
                device_id_type=pl.DeviceIdType.MESH,
            )
            rdma.start()
            rdma.wait()

            @pl.when(s < N_DEV - 1)
            def _():
                out_ref[pl.ds(recv_c * chunk, chunk), :] = (
                    out_ref[pl.ds(recv_c * chunk, chunk), :] + recv_buf[slot]
                )

            @pl.when(s >= N_DEV - 1)
            def _():
                out_ref[pl.ds(recv_c * chunk, chunk), :] = recv_buf[slot]

            pl.semaphore_signal(credit_sem, inc=1, device_id=(left,),
                                device_id_type=pl.DeviceIdType.MESH)
            return carry

        lax.fori_loop(0, N_STEPS, step, 0)

        pl.semaphore_wait(credit_sem, 2)

    return pl.pallas_call(
        body,
        out_shape=jax.ShapeDtypeStruct((m, d), jnp.float32),
        in_specs=[pl.BlockSpec(memory_space=pltpu.VMEM)] * 4,
        out_specs=pl.BlockSpec(memory_space=pltpu.VMEM),
        scratch_shapes=[
            pltpu.VMEM((2, m // N_DEV, d), jnp.float32),
            pltpu.SemaphoreType.DMA((2,)),
            pltpu.SemaphoreType.DMA((2,)),
            pltpu.SemaphoreType.REGULAR,
        ],
        compiler_params=pltpu.CompilerParams(collective_id=0),
    )(x, Wg, Wu, Wd)


# device time: 158407 ns/iter; 1.8589x vs baseline; 1.8589x over previous
import functools

import jax
import jax.numpy as jnp
from jax import lax
from jax.experimental import pallas as pl
from jax.experimental.pallas import tpu as pltpu

N_DEV = 32
P_RING = 8
Z_RING = 4
K_CHUNKS = 4


def kernel(x, Wg, Wu, Wd):
    m, d = x.shape
    h_per = Wg.shape[1]
    blk = m // P_RING
    sub = blk // Z_RING
    kc = h_per // K_CHUNKS

    def body(x_ref, wg_ref, wu_ref, wd_ref, out_ref,
             p1_recv, p2_recv, p3_recv,
             p1_ssem, p1_rsem, p2_ssem, p2_rsem, p3_ssem, p3_rsem):
        my = lax.axis_index("i")
        z = my // P_RING
        p = lax.rem(my, P_RING)
        p_right = z * P_RING + lax.rem(p + 1, P_RING)
        p_left = z * P_RING + lax.rem(p + P_RING - 1, P_RING)
        z_up = lax.rem(z + 1, Z_RING) * P_RING + p
        z_down = lax.rem(z + Z_RING - 1, Z_RING) * P_RING + p
        partners = [p_left, p_right, z_up, z_down]

        barrier_sem = pltpu.get_barrier_semaphore()
        for nbr in partners:
            pl.semaphore_signal(barrier_sem, inc=1, device_id=(nbr,),
                                device_id_type=pl.DeviceIdType.MESH)
        pl.semaphore_wait(barrier_sem, len(partners))

        for k in range(K_CHUNKS):
            sl = slice(k * kc, (k + 1) * kc)
            g = jnp.dot(x_ref[...], wg_ref[:, sl],
                        preferred_element_type=jnp.float32)
            u = jnp.dot(x_ref[...], wu_ref[:, sl],
                        preferred_element_type=jnp.float32)
            hblk = g * (u / (1.0 + jnp.exp(-u)))
            ppart = jnp.dot(hblk, wd_ref[sl, :],
                            preferred_element_type=jnp.float32)
            if k == 0:
                out_ref[...] = ppart
            else:
                out_ref[...] = out_ref[...] + ppart

        for s in range(P_RING - 1):
            send_b = lax.rem(p - s + P_RING, P_RING)
            recv_b = lax.rem(p - s - 1 + P_RING, P_RING)
            rdma = pltpu.make_async_remote_copy(
                src_ref=out_ref.at[pl.ds(send_b * blk, blk), :],
                dst_ref=p1_recv.at[s],
                send_sem=p1_ssem.at[s], recv_sem=p1_rsem.at[s],
                device_id=(p_right,), device_id_type=pl.DeviceIdType.MESH,
            )
            rdma.start()
            rdma.wait()
            out_ref[pl.ds(recv_b * blk, blk), :] = (
                out_ref[pl.ds(recv_b * blk, blk), :] + p1_recv[s]
            )

        q = lax.rem(p + 1, P_RING)
        base = q * blk

        for t in range(2 * (Z_RING - 1)):
            send_c = lax.rem(z - t + 2 * Z_RING, Z_RING)
            recv_c = lax.rem(z - t - 1 + 2 * Z_RING, Z_RING)
            rdma = pltpu.make_async_remote_copy(
                src_ref=out_ref.at[pl.ds(base + send_c * sub, sub), :],
                dst_ref=p2_recv.at[t],
                send_sem=p2_ssem.at[t], recv_sem=p2_rsem.at[t],
                device_id=(z_up,), device_id_type=pl.DeviceIdType.MESH,
            )
            rdma.start()
            rdma.wait()
            if t < Z_RING - 1:
                out_ref[pl.ds(base + recv_c * sub, sub), :] = (
                    out_ref[pl.ds(base + recv_c * sub, sub), :] + p2_recv[t]
                )
            else:
                out_ref[pl.ds(base + recv_c * sub, sub), :] = p2_recv[t]

        for s in range(P_RING - 1):
            send_b = lax.rem(p + 1 - s + P_RING, P_RING)
            recv_b = lax.rem(p - s + P_RING, P_RING)
            rdma = pltpu.make_async_remote_copy(
                src_ref=out_ref.at[pl.ds(send_b * blk, blk), :],
                dst_ref=p3_recv.at[s],
                send_sem=p3_ssem.at[s], recv_sem=p3_rsem.at[s],
                device_id=(p_right,), device_id_type=pl.DeviceIdType.MESH,
            )
            rdma.start()
            rdma.wait()
            out_ref[pl.ds(recv_b * blk, blk), :] = p3_recv[s]

        @functools.partial(pl.run_scoped,
                           exit_sem=pltpu.SemaphoreType.REGULAR)
        def _(exit_sem):
            for nbr in partners:
                pl.semaphore_signal(exit_sem, inc=1, device_id=(nbr,),
                                    device_id_type=pl.DeviceIdType.MESH)
            pl.semaphore_wait(exit_sem, len(partners))

    n_p = P_RING - 1
    n_z = 2 * (Z_RING - 1)
    return pl.pallas_call(
        body,
        out_shape=jax.ShapeDtypeStruct((m, d), jnp.float32),
        in_specs=[pl.BlockSpec(memory_space=pltpu.VMEM)] * 4,
        out_specs=pl.BlockSpec(memory_space=pltpu.VMEM),
        scratch_shapes=[
            pltpu.VMEM((n_p, blk, d), jnp.float32),
            pltpu.VMEM((n_z, sub, d), jnp.float32),
            pltpu.VMEM((n_p, blk, d), jnp.float32),
            pltpu.SemaphoreType.DMA((n_p,)),
            pltpu.SemaphoreType.DMA((n_p,)),
            pltpu.SemaphoreType.DMA((n_z,)),
            pltpu.SemaphoreType.DMA((n_z,)),
            pltpu.SemaphoreType.DMA((n_p,)),
            pltpu.SemaphoreType.DMA((n_p,)),
        ],
        compiler_params=pltpu.CompilerParams(collective_id=0),
    )(x, Wg, Wu, Wd)


# device time: 145456 ns/iter; 2.0244x vs baseline; 1.0890x over previous
import functools

import jax
import jax.numpy as jnp
from jax import lax
from jax.experimental import pallas as pl
from jax.experimental.pallas import tpu as pltpu

N_DEV = 32
P_RING = 8
Z_RING = 4
K_CHUNKS = 4


def kernel(x, Wg, Wu, Wd):
    m, d = x.shape
    h_per = Wg.shape[1]
    blk = m // P_RING
    sub = blk // Z_RING
    kc = h_per // K_CHUNKS

    def body(x_ref, wg_ref, wu_ref, wd_ref, out_ref,
             p1_recv, p2_recv, p3_recv,
             p1_ssem, p1_rsem, p2_ssem, p2_rsem, p3_ssem, p3_rsem):
        my = lax.axis_index("i")
        z = my // P_RING
        p = lax.rem(my, P_RING)
        p_right = z * P_RING + lax.rem(p + 1, P_RING)
        p_left = z * P_RING + lax.rem(p + P_RING - 1, P_RING)
        z_up = lax.rem(z + 1, Z_RING) * P_RING + p
        z_down = lax.rem(z + Z_RING - 1, Z_RING) * P_RING + p
        partners = [p_left, p_right, z_up, z_down]

        barrier_sem = pltpu.get_barrier_semaphore()
        for nbr in partners:
            pl.semaphore_signal(barrier_sem, inc=1, device_id=(nbr,),
                                device_id_type=pl.DeviceIdType.MESH)
        pl.semaphore_wait(barrier_sem, len(partners))

        def compute_block(b):
            rows = pl.ds(b * blk, blk)
            xb = x_ref[rows, :]
            g = jnp.dot(xb, wg_ref[...], preferred_element_type=jnp.float32)
            u = jnp.dot(xb, wu_ref[...], preferred_element_type=jnp.float32)
            hb = g * (u / (1.0 + jnp.exp(-u)))
            out_ref[rows, :] = jnp.dot(hb, wd_ref[...],
                                       preferred_element_type=jnp.float32)

        compute_block(p)

        for s in range(P_RING - 1):
            send_b = lax.rem(p - s + P_RING, P_RING)
            recv_b = lax.rem(p - s - 1 + P_RING, P_RING)
            rdma = pltpu.make_async_remote_copy(
                src_ref=out_ref.at[pl.ds(send_b * blk, blk), :],
                dst_ref=p1_recv.at[s],
                send_sem=p1_ssem.at[s], recv_sem=p1_rsem.at[s],
                device_id=(p_right,), device_id_type=pl.DeviceIdType.MESH,
            )
            rdma.start()
            compute_block(recv_b)
            rdma.wait()
            out_ref[pl.ds(recv_b * blk, blk), :] = (
                out_ref[pl.ds(recv_b * blk, blk), :] + p1_recv[s]
            )

        q = lax.rem(p + 1, P_RING)
        base = q * blk

        for t in range(2 * (Z_RING - 1)):
            send_c = lax.rem(z - t + 2 * Z_RING, Z_RING)
            recv_c = lax.rem(z - t - 1 + 2 * Z_RING, Z_RING)
            rdma = pltpu.make_async_remote_copy(
                src_ref=out_ref.at[pl.ds(base + send_c * sub, sub), :],
                dst_ref=p2_recv.at[t],
                send_sem=p2_ssem.at[t], recv_sem=p2_rsem.at[t],
                device_id=(z_up,), device_id_type=pl.DeviceIdType.MESH,
            )
            rdma.start()
            rdma.wait()
            if t < Z_RING - 1:
                out_ref[pl.ds(base + recv_c * sub, sub), :] = (
                    out_ref[pl.ds(base + recv_c * sub, sub), :] + p2_recv[t]
                )
            else:
                out_ref[pl.ds(base + recv_c * sub, sub), :] = p2_recv[t]

        for s in range(P_RING - 1):
            src = out_ref.at[pl.ds(base, blk), :] if s == 0 else p3_recv.at[s - 1]
            rdma = pltpu.make_async_remote_copy(
                src_ref=src,
                dst_ref=p3_recv.at[s],
                send_sem=p3_ssem.at[s], recv_sem=p3_rsem.at[s],
                device_id=(p_right,), device_id_type=pl.DeviceIdType.MESH,
            )
            rdma.start()
            if s > 0:
                prev_b = lax.rem(p - s + 1 + P_RING, P_RING)
                out_ref[pl.ds(prev_b * blk, blk), :] = p3_recv[s - 1]
            rdma.wait()
        last_b = lax.rem(p - P_RING + 2 + P_RING, P_RING)
        out_ref[pl.ds(last_b * blk, blk), :] = p3_recv[P_RING - 2]

        @functools.partial(pl.run_scoped,
                           exit_sem=pltpu.SemaphoreType.REGULAR)
        def _(exit_sem):
            for nbr in partners:
                pl.semaphore_signal(exit_sem, inc=1, device_id=(nbr,),
                                    device_id_type=pl.DeviceIdType.MESH)
            pl.semaphore_wait(exit_sem, len(partners))

    n_p = P_RING - 1
    n_z = 2 * (Z_RING - 1)
    return pl.pallas_call(
        body,
        out_shape=jax.ShapeDtypeStruct((m, d), jnp.float32),
        in_specs=[pl.BlockSpec(memory_space=pltpu.VMEM)] * 4,
        out_specs=pl.BlockSpec(memory_space=pltpu.VMEM),
        scratch_shapes=[
            pltpu.VMEM((n_p, blk, d), jnp.float32),
            pltpu.VMEM((n_z, sub, d), jnp.float32),
            pltpu.VMEM((n_p, blk, d), jnp.float32),
            pltpu.SemaphoreType.DMA((n_p,)),
            pltpu.SemaphoreType.DMA((n_p,)),
            pltpu.SemaphoreType.DMA((n_z,)),
            pltpu.SemaphoreType.DMA((n_z,)),
            pltpu.SemaphoreType.DMA((n_p,)),
            pltpu.SemaphoreType.DMA((n_p,)),
        ],
        compiler_params=pltpu.CompilerParams(collective_id=0),
    )(x, Wg, Wu, Wd)


# device time: 109687 ns/iter; 2.6845x vs baseline; 1.3261x over previous
import functools

import jax
import jax.numpy as jnp
from jax import lax
from jax.experimental import pallas as pl
from jax.experimental.pallas import tpu as pltpu

N_DEV = 32
P_RING = 8
Z_RING = 4


def kernel(x, Wg, Wu, Wd):
    m, d = x.shape
    h_per = Wg.shape[1]
    blk = m // P_RING
    sub = blk // Z_RING

    def body(x_ref, wg_ref, wu_ref, wd_ref, out_ref,
             x16, wg16, wu16, wd16,
             p1_send, p1_recv, p2_recv, p3_stage, p3_recv,
             p1_ssem, p1_rsem, p2_ssem, p2_rsem, p3_ssem, p3_rsem):
        my = lax.axis_index("i")
        z = my // P_RING
        p = lax.rem(my, P_RING)
        p_right = z * P_RING + lax.rem(p + 1, P_RING)
        p_left = z * P_RING + lax.rem(p + P_RING - 1, P_RING)
        z_up = lax.rem(z + 1, Z_RING) * P_RING + p
        z_down = lax.rem(z + Z_RING - 1, Z_RING) * P_RING + p
        partners = [p_left, p_right, z_up, z_down]

        barrier_sem = pltpu.get_barrier_semaphore()
        for nbr in partners:
            pl.semaphore_signal(barrier_sem, inc=1, device_id=(nbr,),
                                device_id_type=pl.DeviceIdType.MESH)
        pl.semaphore_wait(barrier_sem, len(partners))

        x16[...] = x_ref[...].astype(jnp.bfloat16)
        wg16[...] = wg_ref[...].astype(jnp.bfloat16)
        wu16[...] = wu_ref[...].astype(jnp.bfloat16)
        wd16[...] = wd_ref[...].astype(jnp.bfloat16)

        def compute_block(b):
            rows = pl.ds(b * blk, blk)
            xb = x16[rows, :]
            g = jnp.dot(xb, wg16[...], preferred_element_type=jnp.float32)
            u = jnp.dot(xb, wu16[...], preferred_element_type=jnp.float32)
            hb = (g * (u / (1.0 + jnp.exp(-u)))).astype(jnp.bfloat16)
            out_ref[rows, :] = jnp.dot(hb, wd16[...],
                                       preferred_element_type=jnp.float32)

        compute_block(p)

        for s in range(P_RING - 1):
            send_b = lax.rem(p - s + P_RING, P_RING)
            recv_b = lax.rem(p - s - 1 + P_RING, P_RING)
            p1_send[s] = out_ref[pl.ds(send_b * blk, blk), :].astype(
                jnp.bfloat16)
            rdma = pltpu.make_async_remote_copy(
                src_ref=p1_send.at[s],
                dst_ref=p1_recv.at[s],
                send_sem=p1_ssem.at[s], recv_sem=p1_rsem.at[s],
                device_id=(p_right,), device_id_type=pl.DeviceIdType.MESH,
            )
            rdma.start()
            compute_block(recv_b)
            rdma.wait()
            out_ref[pl.ds(recv_b * blk, blk), :] = (
                out_ref[pl.ds(recv_b * blk, blk), :]
                + p1_recv[s].astype(jnp.float32)
            )

        q = lax.rem(p + 1, P_RING)
        base = q * blk

        for t in range(2 * (Z_RING - 1)):
            send_c = lax.rem(z - t + 2 * Z_RING, Z_RING)
            recv_c = lax.rem(z - t - 1 + 2 * Z_RING, Z_RING)
            rdma = pltpu.make_async_remote_copy(
                src_ref=out_ref.at[pl.ds(base + send_c * sub, sub), :],
                dst_ref=p2_recv.at[t],
                send_sem=p2_ssem.at[t], recv_sem=p2_rsem.at[t],
                device_id=(z_up,), device_id_type=pl.DeviceIdType.MESH,
            )
            rdma.start()
            rdma.wait()
            if t < Z_RING - 1:
                out_ref[pl.ds(base + recv_c * sub, sub), :] = (
                    out_ref[pl.ds(base + recv_c * sub, sub), :] + p2_recv[t]
                )
            else:
                out_ref[pl.ds(base + recv_c * sub, sub), :] = p2_recv[t]

        p3_stage[...] = out_ref[pl.ds(base, blk), :].astype(jnp.bfloat16)
        for s in range(P_RING - 1):
            src = p3_stage if s == 0 else p3_recv.at[s - 1]
            rdma = pltpu.make_async_remote_copy(
                src_ref=src,
                dst_ref=p3_recv.at[s],
                send_sem=p3_ssem.at[s], recv_sem=p3_rsem.at[s],
                device_id=(p_right,), device_id_type=pl.DeviceIdType.MESH,
            )
            rdma.start()
            if s > 0:
                prev_b = lax.rem(p - s + 1 + P_RING, P_RING)
                out_ref[pl.ds(prev_b * blk, blk), :] = (
                    p3_recv[s - 1].astype(jnp.float32)
                )
            rdma.wait()
        last_b = lax.rem(p - P_RING + 2 + P_RING, P_RING)
        out_ref[pl.ds(last_b * blk, blk), :] = (
            p3_recv[P_RING - 2].astype(jnp.float32)
        )

        @functools.partial(pl.run_scoped,
                           exit_sem=pltpu.SemaphoreType.REGULAR)
        def _(exit_sem):
            for nbr in partners:
                pl.semaphore_signal(exit_sem, inc=1, device_id=(nbr,),
                                    device_id_type=pl.DeviceIdType.MESH)
            pl.semaphore_wait(exit_sem, len(partners))

    n_p = P_RING - 1
    n_z = 2 * (Z_RING - 1)
    return pl.pallas_call(
        body,
        out_shape=jax.ShapeDtypeStruct((m, d), jnp.float32),
        in_specs=[pl.BlockSpec(memory_space=pltpu.VMEM)] * 4,
        out_specs=pl.BlockSpec(memory_space=pltpu.VMEM),
        scratch_shapes=[
            pltpu.VMEM((m, d), jnp.bfloat16),
            pltpu.VMEM((d, h_per), jnp.bfloat16),
            pltpu.VMEM((d, h_per), jnp.bfloat16),
            pltpu.VMEM((h_per, d), jnp.bfloat16),
            pltpu.VMEM((n_p, blk, d), jnp.bfloat16),
            pltpu.VMEM((n_p, blk, d), jnp.bfloat16),
            pltpu.VMEM((n_z, sub, d), jnp.float32),
            pltpu.VMEM((blk, d), jnp.bfloat16),
            pltpu.VMEM((n_p, blk, d), jnp.bfloat16),
            pltpu.SemaphoreType.DMA((n_p,)),
            pltpu.SemaphoreType.DMA((n_p,)),
            pltpu.SemaphoreType.DMA((n_z,)),
            pltpu.SemaphoreType.DMA((n_z,)),
            pltpu.SemaphoreType.DMA((n_p,)),
            pltpu.SemaphoreType.DMA((n_p,)),
        ],
        compiler_params=pltpu.CompilerParams(
            collective_id=0, vmem_limit_bytes=100 * 1024 * 1024),
    )(x, Wg, Wu, Wd)


# device time: 97830 ns/iter; 3.0099x vs baseline; 1.1212x over previous
import functools

import jax
import jax.numpy as jnp
from jax import lax
from jax.experimental import pallas as pl
from jax.experimental.pallas import tpu as pltpu

N_DEV = 32
P_RING = 8
Z_RING = 4


def kernel(x, Wg, Wu, Wd):
    m, d = x.shape
    h_per = Wg.shape[1]
    blk = m // P_RING
    sub = blk // Z_RING

    def body(x_ref, wg_ref, wu_ref, wd_ref, out_ref,
             x16, wg16, wu16, wd16,
             p1_send, p1_recv, p2_send, p2_recv, p3_stage, p3_recv,
             p1_ssem, p1_rsem, p2_ssem, p2_rsem, p3_ssem, p3_rsem):
        my = lax.axis_index("i")
        z = my // P_RING
        p = lax.rem(my, P_RING)
        p_right = z * P_RING + lax.rem(p + 1, P_RING)
        p_left = z * P_RING + lax.rem(p + P_RING - 1, P_RING)
        z_up = lax.rem(z + 1, Z_RING) * P_RING + p
        z_down = lax.rem(z + Z_RING - 1, Z_RING) * P_RING + p
        partners = [p_left, p_right, z_up, z_down]

        barrier_sem = pltpu.get_barrier_semaphore()
        for nbr in partners:
            pl.semaphore_signal(barrier_sem, inc=1, device_id=(nbr,),
                                device_id_type=pl.DeviceIdType.MESH)
        pl.semaphore_wait(barrier_sem, len(partners))

        x16[...] = x_ref[...].astype(jnp.bfloat16)
        wg16[...] = wg_ref[...].astype(jnp.bfloat16)
        wu16[...] = wu_ref[...].astype(jnp.bfloat16)
        wd16[...] = wd_ref[...].astype(jnp.bfloat16)

        def compute_block(b):
            rows = pl.ds(b * blk, blk)
            xb = x16[rows, :]
            g = jnp.dot(xb, wg16[...], preferred_element_type=jnp.float32)
            u = jnp.dot(xb, wu16[...], preferred_element_type=jnp.float32)
            hb = (g * (u / (1.0 + jnp.exp(-u)))).astype(jnp.bfloat16)
            out_ref[rows, :] = jnp.dot(hb, wd16[...],
                                       preferred_element_type=jnp.float32)

        compute_block(p)

        for s in range(P_RING - 1):
            send_b = lax.rem(p - s + P_RING, P_RING)
            recv_b = lax.rem(p - s - 1 + P_RING, P_RING)
            p1_send[s] = out_ref[pl.ds(send_b * blk, blk), :].astype(
                jnp.bfloat16)
            rdma = pltpu.make_async_remote_copy(
                src_ref=p1_send.at[s],
                dst_ref=p1_recv.at[s],
                send_sem=p1_ssem.at[s], recv_sem=p1_rsem.at[s],
                device_id=(p_right,), device_id_type=pl.DeviceIdType.MESH,
            )
            rdma.start()
            compute_block(recv_b)
            rdma.wait()
            out_ref[pl.ds(recv_b * blk, blk), :] = (
                out_ref[pl.ds(recv_b * blk, blk), :]
                + p1_recv[s].astype(jnp.float32)
            )

        q = lax.rem(p + 1, P_RING)
        base = q * blk

        for t in range(2 * (Z_RING - 1)):
            send_c = lax.rem(z - t + 2 * Z_RING, Z_RING)
            recv_c = lax.rem(z - t - 1 + 2 * Z_RING, Z_RING)
            p2_send[t] = out_ref[pl.ds(base + send_c * sub, sub), :].astype(
                jnp.bfloat16)
            rdma = pltpu.make_async_remote_copy(
                src_ref=p2_send.at[t],
                dst_ref=p2_recv.at[t],
                send_sem=p2_ssem.at[t], recv_sem=p2_rsem.at[t],
                device_id=(z_up,), device_id_type=pl.DeviceIdType.MESH,
            )
            rdma.start()
            rdma.wait()
            if t < Z_RING - 1:
                out_ref[pl.ds(base + recv_c * sub, sub), :] = (
                    out_ref[pl.ds(base + recv_c * sub, sub), :]
                    + p2_recv[t].astype(jnp.float32)
                )
            else:
                out_ref[pl.ds(base + recv_c * sub, sub), :] = (
                    p2_recv[t].astype(jnp.float32)
                )

        hw = d // 2

        def p3_desc(s, h, src_ref):
            return pltpu.make_async_remote_copy(
                src_ref=src_ref,
                dst_ref=p3_recv.at[s, :, pl.ds(h * hw, hw)],
                send_sem=p3_ssem.at[s, h], recv_sem=p3_rsem.at[s, h],
                device_id=(p_right,), device_id_type=pl.DeviceIdType.MESH,
            )

        p3_stage[...] = out_ref[pl.ds(base, blk), :].astype(jnp.bfloat16)
        for h in range(2):
            p3_desc(0, h, p3_stage.at[:, pl.ds(h * hw, hw)]).start()
        for s in range(1, P_RING - 1):
            for h in range(2):
                prev = p3_desc(s - 1, h,
                               p3_recv.at[s - 1, :, pl.ds(h * hw, hw)])
                prev.wait_recv()
                p3_desc(s, h,
                        p3_recv.at[s - 1, :, pl.ds(h * hw, hw)]).start()
                prev.wait_send()
            prev_b = lax.rem(p - s + 1 + P_RING, P_RING)
            out_ref[pl.ds(prev_b * blk, blk), :] = (
                p3_recv[s - 1].astype(jnp.float32)
            )
        for h in range(2):
            last = p3_desc(P_RING - 2, h,
                           p3_recv.at[P_RING - 2, :, pl.ds(h * hw, hw)])
            last.wait_recv()
            last.wait_send()
        last_b = lax.rem(p - P_RING + 2 + P_RING, P_RING)
        out_ref[pl.ds(last_b * blk, blk), :] = (
            p3_recv[P_RING - 2].astype(jnp.float32)
        )

        @functools.partial(pl.run_scoped,
                           exit_sem=pltpu.SemaphoreType.REGULAR)
        def _(exit_sem):
            for nbr in partners:
                pl.semaphore_signal(exit_sem, inc=1, device_id=(nbr,),
                                    device_id_type=pl.DeviceIdType.MESH)
            pl.semaphore_wait(exit_sem, len(partners))

    n_p = P_RING - 1
    n_z = 2 * (Z_RING - 1)
    return pl.pallas_call(
        body,
        out_shape=jax.ShapeDtypeStruct((m, d), jnp.float32),
        in_specs=[pl.BlockSpec(memory_space=pltpu.VMEM)] * 4,
        out_specs=pl.BlockSpec(memory_space=pltpu.VMEM),
        scratch_shapes=[
            pltpu.VMEM((m, d), jnp.bfloat16),
            pltpu.VMEM((d, h_per), jnp.bfloat16),
            pltpu.VMEM((d, h_per), jnp.bfloat16),
            pltpu.VMEM((h_per, d), jnp.bfloat16),
            pltpu.VMEM((n_p, blk, d), jnp.bfloat16),
            pltpu.VMEM((n_p, blk, d), jnp.bfloat16),
            pltpu.VMEM((n_z, sub, d), jnp.bfloat16),
            pltpu.VMEM((n_z, sub, d), jnp.bfloat16),
            pltpu.VMEM((blk, d), jnp.bfloat16),
            pltpu.VMEM((n_p, blk, d), jnp.bfloat16),
            pltpu.SemaphoreType.DMA((n_p,)),
            pltpu.SemaphoreType.DMA((n_p,)),
            pltpu.SemaphoreType.DMA((n_z,)),
            pltpu.SemaphoreType.DMA((n_z,)),
            pltpu.SemaphoreType.DMA((n_p, 2)),
            pltpu.SemaphoreType.DMA((n_p, 2)),
        ],
        compiler_params=pltpu.CompilerParams(
            collective_id=0, vmem_limit_bytes=100 * 1024 * 1024),
    )(x, Wg, Wu, Wd)


# device time: 95344 ns/iter; 3.0884x vs baseline; 1.0261x over previous
import functools

import jax
import jax.numpy as jnp
from jax import lax
from jax.experimental import pallas as pl
from jax.experimental.pallas import tpu as pltpu

N_DEV = 32
P_RING = 8
Z_RING = 4


def kernel(x, Wg, Wu, Wd):
    m, d = x.shape
    h_per = Wg.shape[1]
    blk = m // P_RING
    sub = blk // Z_RING

    def body(x_ref, wg_ref, wu_ref, wd_ref, out_ref,
             x16, wg16, wu16, wd16,
             p1_send, p1_recv, p2_send, p2_recv, p3_stage, p3_recv,
             p1_ssem, p1_rsem, p2_ssem, p2_rsem, p3_ssem, p3_rsem):
        my = lax.axis_index("i")
        z = my // P_RING
        p = lax.rem(my, P_RING)
        p_right = z * P_RING + lax.rem(p + 1, P_RING)
        p_left = z * P_RING + lax.rem(p + P_RING - 1, P_RING)
        z_up = lax.rem(z + 1, Z_RING) * P_RING + p
        z_down = lax.rem(z + Z_RING - 1, Z_RING) * P_RING + p
        partners = [p_left, p_right, z_up, z_down]

        barrier_sem = pltpu.get_barrier_semaphore()
        for nbr in partners:
            pl.semaphore_signal(barrier_sem, inc=1, device_id=(nbr,),
                                device_id_type=pl.DeviceIdType.MESH)
        pl.semaphore_wait(barrier_sem, len(partners))

        x16[...] = x_ref[...].astype(jnp.bfloat16)
        wg16[...] = wg_ref[...].astype(jnp.bfloat16)
        wu16[...] = wu_ref[...].astype(jnp.bfloat16)
        wd16[...] = wd_ref[...].astype(jnp.bfloat16)

        def compute_block(b):
            rows = pl.ds(b * blk, blk)
            xb = x16[rows, :]
            g = jnp.dot(xb, wg16[...], preferred_element_type=jnp.float32)
            u = jnp.dot(xb, wu16[...], preferred_element_type=jnp.float32)
            hb = (g * (u / (1.0 + jnp.exp(-u)))).astype(jnp.bfloat16)
            out_ref[rows, :] = jnp.dot(hb, wd16[...],
                                       preferred_element_type=jnp.float32)

        compute_block(p)

        for s in range(P_RING - 1):
            send_b = lax.rem(p - s + P_RING, P_RING)
            recv_b = lax.rem(p - s - 1 + P_RING, P_RING)
            p1_send[s] = out_ref[pl.ds(send_b * blk, blk), :].astype(
                jnp.bfloat16)
            rdma = pltpu.make_async_remote_copy(
                src_ref=p1_send.at[s],
                dst_ref=p1_recv.at[s],
                send_sem=p1_ssem.at[s], recv_sem=p1_rsem.at[s],
                device_id=(p_right,), device_id_type=pl.DeviceIdType.MESH,
            )
            rdma.start()
            compute_block(recv_b)
            rdma.wait()
            out_ref[pl.ds(recv_b * blk, blk), :] = (
                out_ref[pl.ds(recv_b * blk, blk), :]
                + p1_recv[s].astype(jnp.float32)
            )

        q = lax.rem(p + 1, P_RING)
        base = q * blk

        hw2 = d // 2
        n_z_steps = 2 * (Z_RING - 1)

        def p2_desc(t, h, src_ref):
            return pltpu.make_async_remote_copy(
                src_ref=src_ref,
                dst_ref=p2_recv.at[t, :, pl.ds(h * hw2, hw2)],
                send_sem=p2_ssem.at[t, h], recv_sem=p2_rsem.at[t, h],
                device_id=(z_up,), device_id_type=pl.DeviceIdType.MESH,
            )

        def p2_consume(t, h):
            rc = lax.rem(z - t - 1 + 2 * Z_RING, Z_RING)
            rows = pl.ds(base + rc * sub, sub)
            cols = pl.ds(h * hw2, hw2)
            if t < Z_RING - 1:
                out_ref[rows, cols] = (
                    out_ref[rows, cols] + p2_recv[t, :, h * hw2:(h + 1) * hw2]
                    .astype(jnp.float32)
                )
            else:
                out_ref[rows, cols] = (
                    p2_recv[t, :, h * hw2:(h + 1) * hw2].astype(jnp.float32)
                )

        for t in range(n_z_steps):
            send_c = lax.rem(z - t + 2 * Z_RING, Z_RING)
            for h in range(2):
                cols = pl.ds(h * hw2, hw2)
                if t > 0:
                    prev = p2_desc(t - 1, h, p2_send.at[t - 1, :, cols])
                    prev.wait_recv()
                    p2_consume(t - 1, h)
                    prev.wait_send()
                p2_send[t, :, h * hw2:(h + 1) * hw2] = (
                    out_ref[pl.ds(base + send_c * sub, sub), cols]
                    .astype(jnp.bfloat16)
                )
                p2_desc(t, h, p2_send.at[t, :, cols]).start()
        for h in range(2):
            cols = pl.ds(h * hw2, hw2)
            last = p2_desc(n_z_steps - 1, h, p2_send.at[n_z_steps - 1, :, cols])
            last.wait_recv()
            p2_consume(n_z_steps - 1, h)
            last.wait_send()

        n_q = 4
        hw = d // n_q

        def p3_desc(s, h, src_ref):
            return pltpu.make_async_remote_copy(
                src_ref=src_ref,
                dst_ref=p3_recv.at[s, :, pl.ds(h * hw, hw)],
                send_sem=p3_ssem.at[s, h], recv_sem=p3_rsem.at[s, h],
                device_id=(p_right,), device_id_type=pl.DeviceIdType.MESH,
            )

        p3_stage[...] = out_ref[pl.ds(base, blk), :].astype(jnp.bfloat16)
        for h in range(n_q):
            p3_desc(0, h, p3_stage.at[:, pl.ds(h * hw, hw)]).start()
        for s in range(1, P_RING - 1):
            for h in range(n_q):
                prev = p3_desc(s - 1, h,
                               p3_recv.at[s - 1, :, pl.ds(h * hw, hw)])
                prev.wait_recv()
                p3_desc(s, h,
                        p3_recv.at[s - 1, :, pl.ds(h * hw, hw)]).start()
                prev.wait_send()
            prev_b = lax.rem(p - s + 1 + P_RING, P_RING)
            out_ref[pl.ds(prev_b * blk, blk), :] = (
                p3_recv[s - 1].astype(jnp.float32)
            )
        for h in range(n_q):
            last = p3_desc(P_RING - 2, h,
                           p3_recv.at[P_RING - 2, :, pl.ds(h * hw, hw)])
            last.wait_recv()
            last.wait_send()
        last_b = lax.rem(p - P_RING + 2 + P_RING, P_RING)
        out_ref[pl.ds(last_b * blk, blk), :] = (
            p3_recv[P_RING - 2].astype(jnp.float32)
        )

        @functools.partial(pl.run_scoped,
                           exit_sem=pltpu.SemaphoreType.REGULAR)
        def _(exit_sem):
            for nbr in partners:
                pl.semaphore_signal(exit_sem, inc=1, device_id=(nbr,),
                                    device_id_type=pl.DeviceIdType.MESH)
            pl.semaphore_wait(exit_sem, len(partners))

    n_p = P_RING - 1
    n_z = 2 * (Z_RING - 1)
    return pl.pallas_call(
        body,
        out_shape=jax.ShapeDtypeStruct((m, d), jnp.float32),
        in_specs=[pl.BlockSpec(memory_space=pltpu.VMEM)] * 4,
        out_specs=pl.BlockSpec(memory_space=pltpu.VMEM),
        scratch_shapes=[
            pltpu.VMEM((m, d), jnp.bfloat16),
            pltpu.VMEM((d, h_per), jnp.bfloat16),
            pltpu.VMEM((d, h_per), jnp.bfloat16),
            pltpu.VMEM((h_per, d), jnp.bfloat16),
            pltpu.VMEM((n_p, blk, d), jnp.bfloat16),
            pltpu.VMEM((n_p, blk, d), jnp.bfloat16),
            pltpu.VMEM((n_z, sub, d), jnp.bfloat16),
            pltpu.VMEM((n_z, sub, d), jnp.bfloat16),
            pltpu.VMEM((blk, d), jnp.bfloat16),
            pltpu.VMEM((n_p, blk, d), jnp.bfloat16),
            pltpu.SemaphoreType.DMA((n_p,)),
            pltpu.SemaphoreType.DMA((n_p,)),
            pltpu.SemaphoreType.DMA((n_z, 2)),
            pltpu.SemaphoreType.DMA((n_z, 2)),
            pltpu.SemaphoreType.DMA((n_p, 4)),
            pltpu.SemaphoreType.DMA((n_p, 4)),
        ],
        compiler_params=pltpu.CompilerParams(
            collective_id=0, vmem_limit_bytes=100 * 1024 * 1024),
    )(x, Wg, Wu, Wd)


# device time: 86776 ns/iter; 3.3933x vs baseline; 1.0987x over previous
import functools

import jax
import jax.numpy as jnp
from jax import lax
from jax.experimental import pallas as pl
from jax.experimental.pallas import tpu as pltpu

N_DEV = 32
P_RING = 8
Z_RING = 4


def kernel(x, Wg, Wu, Wd):
    m, d = x.shape
    h_per = Wg.shape[1]
    blk = m // P_RING
    sub = blk // Z_RING

    def body(x_ref, wg_ref, wu_ref, wd_ref, out_ref,
             x16, wg16, wu16, wd16,
             p1_send, p1_recv, p2_send, p2_recv, p3_stage, p3_recv,
             p1_ssem, p1_rsem, p2_ssem, p2_rsem, p3_ssem, p3_rsem):
        my = lax.axis_index("i")
        z = my // P_RING
        p = lax.rem(my, P_RING)
        p_right = z * P_RING + lax.rem(p + 1, P_RING)
        p_left = z * P_RING + lax.rem(p + P_RING - 1, P_RING)
        z_up = lax.rem(z + 1, Z_RING) * P_RING + p
        z_down = lax.rem(z + Z_RING - 1, Z_RING) * P_RING + p
        partners = [p_left, p_right, z_up, z_down]

        barrier_sem = pltpu.get_barrier_semaphore()
        for nbr in partners:
            pl.semaphore_signal(barrier_sem, inc=1, device_id=(nbr,),
                                device_id_type=pl.DeviceIdType.MESH)
        pl.semaphore_wait(barrier_sem, len(partners))

        x16[...] = x_ref[...].astype(jnp.bfloat16)
        wg16[...] = wg_ref[...].astype(jnp.bfloat16)
        wu16[...] = wu_ref[...].astype(jnp.bfloat16)
        wd16[...] = wd_ref[...].astype(jnp.bfloat16)

        def compute_block(b):
            rows = pl.ds(b * blk, blk)
            xb = x16[rows, :]
            g = jnp.dot(xb, wg16[...], preferred_element_type=jnp.float32)
            u = jnp.dot(xb, wu16[...], preferred_element_type=jnp.float32)
            hb = (g * (u / (1.0 + jnp.exp(-u)))).astype(jnp.bfloat16)
            out_ref[rows, :] = jnp.dot(hb, wd16[...],
                                       preferred_element_type=jnp.float32)

        compute_block(p)

        n_q1 = 4
        qw = d // n_q1

        def p1_desc(s, h, src_ref):
            return pltpu.make_async_remote_copy(
                src_ref=src_ref,
                dst_ref=p1_recv.at[s, :, pl.ds(h * qw, qw)],
                send_sem=p1_ssem.at[s, h], recv_sem=p1_rsem.at[s, h],
                device_id=(p_right,), device_id_type=pl.DeviceIdType.MESH,
            )

        for s in range(P_RING - 1):
            send_b = lax.rem(p - s + P_RING, P_RING)
            recv_b = lax.rem(p - s - 1 + P_RING, P_RING)
            srows = pl.ds(send_b * blk, blk)
            for h in range(n_q1):
                cols = pl.ds(h * qw, qw)
                if s > 0:
                    prev = p1_desc(s - 1, h, p1_send.at[s - 1, :, cols])
                    prev.wait_recv()
                    out_ref[srows, cols] = (
                        out_ref[srows, cols]
                        + p1_recv[s - 1, :, h * qw:(h + 1) * qw]
                        .astype(jnp.float32)
                    )
                    prev.wait_send()
                p1_send[s, :, h * qw:(h + 1) * qw] = (
                    out_ref[srows, cols].astype(jnp.bfloat16)
                )
                p1_desc(s, h, p1_send.at[s, :, cols]).start()
            compute_block(recv_b)
        qrows = pl.ds(lax.rem(p + 1, P_RING) * blk, blk)
        for h in range(n_q1):
            cols = pl.ds(h * qw, qw)
            last = p1_desc(P_RING - 2, h, p1_send.at[P_RING - 2, :, cols])
            last.wait_recv()
            out_ref[qrows, cols] = (
                out_ref[qrows, cols]
                + p1_recv[P_RING - 2, :, h * qw:(h + 1) * qw]
                .astype(jnp.float32)
            )
            last.wait_send()

        q = lax.rem(p + 1, P_RING)
        base = q * blk

        hw2 = d // 2
        n_z_steps = 2 * (Z_RING - 1)

        def p2_desc(t, h, src_ref):
            return pltpu.make_async_remote_copy(
                src_ref=src_ref,
                dst_ref=p2_recv.at[t, :, pl.ds(h * hw2, hw2)],
                send_sem=p2_ssem.at[t, h], recv_sem=p2_rsem.at[t, h],
                device_id=(z_up,), device_id_type=pl.DeviceIdType.MESH,
            )

        def p2_consume(t, h):
            rc = lax.rem(z - t - 1 + 2 * Z_RING, Z_RING)
            rows = pl.ds(base + rc * sub, sub)
            cols = pl.ds(h * hw2, hw2)
            if t < Z_RING - 1:
                out_ref[rows, cols] = (
                    out_ref[rows, cols] + p2_recv[t, :, h * hw2:(h + 1) * hw2]
                    .astype(jnp.float32)
                )
            else:
                out_ref[rows, cols] = (
                    p2_recv[t, :, h * hw2:(h + 1) * hw2].astype(jnp.float32)
                )

        for t in range(n_z_steps):
            send_c = lax.rem(z - t + 2 * Z_RING, Z_RING)
            for h in range(2):
                cols = pl.ds(h * hw2, hw2)
                if t > 0:
                    prev = p2_desc(t - 1, h, p2_send.at[t - 1, :, cols])
                    prev.wait_recv()
                    p2_consume(t - 1, h)
                    prev.wait_send()
                p2_send[t, :, h * hw2:(h + 1) * hw2] = (
                    out_ref[pl.ds(base + send_c * sub, sub), cols]
                    .astype(jnp.bfloat16)
                )
                p2_desc(t, h, p2_send.at[t, :, cols]).start()
        for h in range(2):
            cols = pl.ds(h * hw2, hw2)
            last = p2_desc(n_z_steps - 1, h, p2_send.at[n_z_steps - 1, :, cols])
            last.wait_recv()
            p2_consume(n_z_steps - 1, h)
            last.wait_send()

        n_q = 4
        hw = d // n_q

        def p3_desc(s, h, src_ref):
            return pltpu.make_async_remote_copy(
                src_ref=src_ref,
                dst_ref=p3_recv.at[s, :, pl.ds(h * hw, hw)],
                send_sem=p3_ssem.at[s, h], recv_sem=p3_rsem.at[s, h],
                device_id=(p_right,), device_id_type=pl.DeviceIdType.MESH,
            )

        p3_stage[...] = out_ref[pl.ds(base, blk), :].astype(jnp.bfloat16)
        for h in range(n_q):
            p3_desc(0, h, p3_stage.at[:, pl.ds(h * hw, hw)]).start()
        for s in range(1, P_RING - 1):
            for h in range(n_q):
                prev = p3_desc(s - 1, h,
                               p3_recv.at[s - 1, :, pl.ds(h * hw, hw)])
                prev.wait_recv()
                p3_desc(s, h,
                        p3_recv.at[s - 1, :, pl.ds(h * hw, hw)]).start()
                prev.wait_send()
            prev_b = lax.rem(p - s + 1 + P_RING, P_RING)
            out_ref[pl.ds(prev_b * blk, blk), :] = (
                p3_recv[s - 1].astype(jnp.float32)
            )
        for h in range(n_q):
            last = p3_desc(P_RING - 2, h,
                           p3_recv.at[P_RING - 2, :, pl.ds(h * hw, hw)])
            last.wait_recv()
            last.wait_send()
        last_b = lax.rem(p - P_RING + 2 + P_RING, P_RING)
        out_ref[pl.ds(last_b * blk, blk), :] = (
            p3_recv[P_RING - 2].astype(jnp.float32)
        )

        @functools.partial(pl.run_scoped,
                           exit_sem=pltpu.SemaphoreType.REGULAR)
        def _(exit_sem):
            for nbr in partners:
                pl.semaphore_signal(exit_sem, inc=1, device_id=(nbr,),
                                    device_id_type=pl.DeviceIdType.MESH)
            pl.semaphore_wait(exit_sem, len(partners))

    n_p = P_RING - 1
    n_z = 2 * (Z_RING - 1)
    return pl.pallas_call(
        body,
        out_shape=jax.ShapeDtypeStruct((m, d), jnp.float32),
        in_specs=[pl.BlockSpec(memory_space=pltpu.VMEM)] * 4,
        out_specs=pl.BlockSpec(memory_space=pltpu.VMEM),
        scratch_shapes=[
            pltpu.VMEM((m, d), jnp.bfloat16),
            pltpu.VMEM((d, h_per), jnp.bfloat16),
            pltpu.VMEM((d, h_per), jnp.bfloat16),
            pltpu.VMEM((h_per, d), jnp.bfloat16),
            pltpu.VMEM((n_p, blk, d), jnp.bfloat16),
            pltpu.VMEM((n_p, blk, d), jnp.bfloat16),
            pltpu.VMEM((n_z, sub, d), jnp.bfloat16),
            pltpu.VMEM((n_z, sub, d), jnp.bfloat16),
            pltpu.VMEM((blk, d), jnp.bfloat16),
            pltpu.VMEM((n_p, blk, d), jnp.bfloat16),
            pltpu.SemaphoreType.DMA((n_p, 4)),
            pltpu.SemaphoreType.DMA((n_p, 4)),
            pltpu.SemaphoreType.DMA((n_z, 2)),
            pltpu.SemaphoreType.DMA((n_z, 2)),
            pltpu.SemaphoreType.DMA((n_p, 4)),
            pltpu.SemaphoreType.DMA((n_p, 4)),
        ],
        compiler_params=pltpu.CompilerParams(
            collective_id=0, vmem_limit_bytes=100 * 1024 * 1024),
    )(x, Wg, Wu, Wd)
